# baseline (speedup 1.0000x reference)
import dataclasses

import jax
import jax.numpy as jnp
from jax import lax
from jax.experimental import pallas as pl
from jax.experimental.pallas import tpu as pltpu
from jax.experimental.pallas import tpu_sc as plsc

N = 10000
D = 128
E = 320000

NUM_CORES = 2
NUM_SUBCORES = 16
HCHUNK = 128
NH = 81
SPAN = NH * HCHUNK
EPC_PAD = NUM_SUBCORES * SPAN
PAD = EPC_PAD - E // NUM_CORES
OWN_ROWS = 1000
ZROWS = 40


def _sc_aggregate(x, src, dst, w):
    mesh = plsc.VectorSubcoreMesh(core_axis_name="c", subcore_axis_name="s")

    @pl.kernel(
        out_type=jax.ShapeDtypeStruct((NUM_CORES, N, D), jnp.float32),
        mesh=mesh,
        scratch_types=[
            pltpu.VMEM_SHARED((N, D), jnp.float32),
        ] + [
            pltpu.VMEM((HCHUNK, D), jnp.float32)
            for _ in range(3)
        ] + [
            pltpu.VMEM((HCHUNK,), jnp.int32)
            for _ in range(6)
        ] + [
            pltpu.VMEM((HCHUNK,), jnp.float32)
            for _ in range(3)
        ] + [
            pltpu.SemaphoreType.DMA,
            pltpu.SemaphoreType.DMA,
            pltpu.SemaphoreType.DMA,
            pltpu.SemaphoreType.DMA,
        ],
    )
    def agg_kernel(x_hbm, src_hbm, dst_hbm, w_hbm, out_hbm,
                   acc, rows0, rows1, rows2,
                   src0, dst0, src1, dst1, src2, dst2, w0, w1, w2,
                   g0, g1, g2, semI):
        cid = lax.axis_index("c")
        sid = lax.axis_index("s")
        rows_v = rows0

        @pl.when(sid < N // OWN_ROWS)
        def _():
            zero16 = jnp.zeros((16,), jnp.float32)
            for r in range(ZROWS):
                for j in range(D // 16):
                    rows_v[r, pl.ds(j * 16, 16)] = zero16
            base_row = pl.multiple_of(sid * OWN_ROWS, 8)

            @pl.loop(0, OWN_ROWS, step=ZROWS)
            def _(t):
                pltpu.sync_copy(rows_v.at[pl.ds(0, ZROWS)],
                                acc.at[pl.ds(base_row + t, ZROWS)])

        plsc.subcore_barrier()

        ebase = pl.multiple_of(cid * EPC_PAD + sid * SPAN, 8)
        bufs = (
            (rows0, src0, dst0, w0, g0),
            (rows1, src1, dst1, w1, g1),
            (rows2, src2, dst2, w2, g2),
        )

        def idx_copies(h, buf):
            _, s_v, d_v, w_v, _ = buf
            b = pl.multiple_of(ebase + h * HCHUNK, 8)
            return (
                pltpu.make_async_copy(src_hbm.at[pl.ds(b, HCHUNK)], s_v, semI),
                pltpu.make_async_copy(dst_hbm.at[pl.ds(b, HCHUNK)], d_v, semI),
                pltpu.make_async_copy(w_hbm.at[pl.ds(b, HCHUNK)], w_v, semI),
            )

        def start_gather(buf):
            rows_b, s_v, _, _, g_v = buf
            pltpu.make_async_copy(x_hbm.at[s_v], rows_b, g_v).start()

        def scale(buf):
            rows_b, _, _, w_v, _ = buf
            @plsc.parallel_loop(0, HCHUNK, step=16)
            def _(g):
                wg = w_v[pl.ds(g, 16)]
                for k in range(16):
                    wi = wg[k]
                    for j in range(D // 16):
                        sl = pl.ds(j * 16, 16)
                        rows_b[g + k, sl] = rows_b[g + k, sl] * wi

        for st in range(2):
            for h in idx_copies(st, bufs[st]):
                h.start()
                h.wait()
            start_gather(bufs[st])

        @pl.loop(0, NH, step=3)
        def _(k):
            for p in range(3):
                h = k + p
                buf = bufs[p]
                nxt = bufs[(p + 2) % 3]
                rows_b, s_v, d_v, w_v, g_v = buf

                @pl.when(h + 2 < NH)
                def _():
                    for hh in idx_copies(h + 2, nxt):
                        hh.start()
                        hh.wait()
                    start_gather(nxt)

                pltpu.make_async_copy(x_hbm.at[s_v], rows_b, g_v).wait()
                scale(buf)
                pltpu.sync_copy(rows_b, acc.at[d_v], add=True)

        plsc.subcore_barrier()

        @pl.when(sid < N // OWN_ROWS)
        def _():
            base_row = pl.multiple_of(sid * OWN_ROWS, 8)
            pltpu.sync_copy(acc.at[pl.ds(base_row, OWN_ROWS)],
                            out_hbm.at[cid].at[pl.ds(base_row, OWN_ROWS)])

    return agg_kernel(x, src, dst, w)


def _tc_mlp(x, partials, W1x, W1a, b1, W2, b2):
    BLK = 2000

    def body(x_ref, p0_ref, p1_ref, W1x_ref, W1a_ref, b1_ref, W2_ref, b2_ref,
             o_ref):
        agg = p0_ref[0] + p1_ref[0]
        h = jnp.dot(x_ref[...], W1x_ref[...], preferred_element_type=jnp.float32)
        h += jnp.dot(agg, W1a_ref[...], preferred_element_type=jnp.float32)
        h = jnp.maximum(h + b1_ref[...], 0.0)
        o_ref[...] = (
            jnp.dot(h, W2_ref[...], preferred_element_type=jnp.float32)
            + b2_ref[...]
        )

    full = lambda i: (0, 0)
    return pl.pallas_call(
        body,
        grid=(N // BLK,),
        in_specs=[
            pl.BlockSpec((BLK, D), lambda i: (i, 0)),
            pl.BlockSpec((1, BLK, D), lambda i: (0, i, 0)),
            pl.BlockSpec((1, BLK, D), lambda i: (1, i, 0)),
            pl.BlockSpec((D, D), full),
            pl.BlockSpec((D, D), full),
            pl.BlockSpec((1, D), full),
            pl.BlockSpec((D, D), full),
            pl.BlockSpec((1, D), full),
        ],
        out_specs=pl.BlockSpec((BLK, D), lambda i: (i, 0)),
        out_shape=jax.ShapeDtypeStruct((N, D), jnp.float32),
    )(x, partials, partials, W1x, W1a, b1, W2, b2)


def kernel(x, edge_index, edge_weight, W1, b1, W2, b2):
    src = edge_index[1].astype(jnp.int32)
    dst = edge_index[0].astype(jnp.int32)
    half = E // NUM_CORES
    zi = jnp.arange(PAD, dtype=jnp.int32)
    zf = jnp.zeros((PAD,), jnp.float32)
    src_p = jnp.concatenate([src[:half], zi, src[half:], zi])
    dst_p = jnp.concatenate([dst[:half], zi, dst[half:], zi])
    w_p = jnp.concatenate([edge_weight[:half], zf, edge_weight[half:], zf])
    partials = _sc_aggregate(x, src_p, dst_p, w_p)
    W1x = W1[:D]
    W1a = W1[D:]
    return _tc_mlp(x, partials, W1x, W1a, b1.reshape(1, D), W2,
                   b2.reshape(1, D))


# speedup vs baseline: 1.5473x; 1.5473x over previous
import dataclasses

import jax
import jax.numpy as jnp
from jax import lax
from jax.experimental import pallas as pl
from jax.experimental.pallas import tpu as pltpu
from jax.experimental.pallas import tpu_sc as plsc

N = 10000
D = 128
E = 320000

NUM_CORES = 2
NUM_SUBCORES = 16
HCHUNK = 112
NH = 90
NI = 6
SPAN = NH * HCHUNK
EPC_PAD = NUM_SUBCORES * SPAN
PAD = EPC_PAD - E // NUM_CORES
OWN_ROWS = 1000
ZROWS = 40


def _sc_aggregate(x, src, dst, w):
    mesh = plsc.VectorSubcoreMesh(core_axis_name="c", subcore_axis_name="s")

    @pl.kernel(
        out_type=jax.ShapeDtypeStruct((NUM_CORES, N, D), jnp.float32),
        mesh=mesh,
        scratch_types=[
            pltpu.VMEM_SHARED((N, D), jnp.float32),
        ] + [
            pltpu.VMEM((HCHUNK, D), jnp.float32)
            for _ in range(3)
        ] + [
            pltpu.VMEM((HCHUNK,), jnp.int32)
            for _ in range(2 * NI)
        ] + [
            pltpu.VMEM((HCHUNK,), jnp.float32)
            for _ in range(NI)
        ] + [
            pltpu.SemaphoreType.DMA
            for _ in range(3 + NI)
        ],
    )
    def agg_kernel(x_hbm, src_hbm, dst_hbm, w_hbm, out_hbm, acc, *bufs_flat):
        rows_ring = bufs_flat[0:3]
        src_ring = bufs_flat[3:3 + NI]
        dst_ring = bufs_flat[3 + NI:3 + 2 * NI]
        w_ring = bufs_flat[3 + 2 * NI:3 + 3 * NI]
        g_sems = bufs_flat[3 + 3 * NI:6 + 3 * NI]
        i_sems = bufs_flat[6 + 3 * NI:6 + 4 * NI]
        cid = lax.axis_index("c")
        sid = lax.axis_index("s")
        rows_v = rows_ring[0]

        @pl.when(sid < N // OWN_ROWS)
        def _():
            zero16 = jnp.zeros((16,), jnp.float32)
            for r in range(ZROWS):
                for j in range(D // 16):
                    rows_v[r, pl.ds(j * 16, 16)] = zero16
            base_row = pl.multiple_of(sid * OWN_ROWS, 8)

            @pl.loop(0, OWN_ROWS, step=ZROWS)
            def _(t):
                pltpu.sync_copy(rows_v.at[pl.ds(0, ZROWS)],
                                acc.at[pl.ds(base_row + t, ZROWS)])

        plsc.subcore_barrier()

        ebase = pl.multiple_of(cid * EPC_PAD + sid * SPAN, 8)

        def idx_copies(h, i):
            b = pl.multiple_of(ebase + h * HCHUNK, 8)
            return (
                pltpu.make_async_copy(
                    src_hbm.at[pl.ds(b, HCHUNK)], src_ring[i], i_sems[i]),
                pltpu.make_async_copy(
                    dst_hbm.at[pl.ds(b, HCHUNK)], dst_ring[i], i_sems[i]),
                pltpu.make_async_copy(
                    w_hbm.at[pl.ds(b, HCHUNK)], w_ring[i], i_sems[i]),
            )

        def start_gather(p, i):
            pltpu.make_async_copy(
                x_hbm.at[src_ring[i]], rows_ring[p], g_sems[p]).start()

        def scale(p, i):
            rows_b, w_v = rows_ring[p], w_ring[i]
            @plsc.parallel_loop(0, HCHUNK, step=16)
            def _(g):
                wg = w_v[pl.ds(g, 16)]
                for k in range(16):
                    wi = wg[k]
                    for j in range(D // 16):
                        sl = pl.ds(j * 16, 16)
                        rows_b[g + k, sl] = rows_b[g + k, sl] * wi

        for st in range(3):
            for hh in idx_copies(st, st):
                hh.start()
        for st in range(2):
            for hh in idx_copies(st, st):
                hh.wait()
            start_gather(st, st)

        @pl.loop(0, NH, step=NI)
        def _(k):
            for q in range(NI):
                h = k + q
                p = q % 3
                i = q
                i2 = (q + 2) % NI
                i3 = (q + 3) % NI
                p2 = (q + 2) % 3

                @pl.when(h + 3 < NH)
                def _():
                    for hh in idx_copies(h + 3, i3):
                        hh.start()

                @pl.when(h + 2 < NH)
                def _():
                    for hh in idx_copies(h + 2, i2):
                        hh.wait()
                    start_gather(p2, i2)

                pltpu.make_async_copy(
                    x_hbm.at[src_ring[i]], rows_ring[p], g_sems[p]).wait()
                scale(p, i)
                pltpu.sync_copy(rows_ring[p], acc.at[dst_ring[i]], add=True)

        plsc.subcore_barrier()

        @pl.when(sid < N // OWN_ROWS)
        def _():
            base_row = pl.multiple_of(sid * OWN_ROWS, 8)
            pltpu.sync_copy(acc.at[pl.ds(base_row, OWN_ROWS)],
                            out_hbm.at[cid].at[pl.ds(base_row, OWN_ROWS)])

    return agg_kernel(x, src, dst, w)


def _tc_mlp(x, partials, W1x, W1a, b1, W2, b2):
    BLK = 2000

    def body(x_ref, p0_ref, p1_ref, W1x_ref, W1a_ref, b1_ref, W2_ref, b2_ref,
             o_ref):
        agg = p0_ref[0] + p1_ref[0]
        h = jnp.dot(x_ref[...], W1x_ref[...], preferred_element_type=jnp.float32)
        h += jnp.dot(agg, W1a_ref[...], preferred_element_type=jnp.float32)
        h = jnp.maximum(h + b1_ref[...], 0.0)
        o_ref[...] = (
            jnp.dot(h, W2_ref[...], preferred_element_type=jnp.float32)
            + b2_ref[...]
        )

    full = lambda i: (0, 0)
    return pl.pallas_call(
        body,
        grid=(N // BLK,),
        in_specs=[
            pl.BlockSpec((BLK, D), lambda i: (i, 0)),
            pl.BlockSpec((1, BLK, D), lambda i: (0, i, 0)),
            pl.BlockSpec((1, BLK, D), lambda i: (1, i, 0)),
            pl.BlockSpec((D, D), full),
            pl.BlockSpec((D, D), full),
            pl.BlockSpec((1, D), full),
            pl.BlockSpec((D, D), full),
            pl.BlockSpec((1, D), full),
        ],
        out_specs=pl.BlockSpec((BLK, D), lambda i: (i, 0)),
        out_shape=jax.ShapeDtypeStruct((N, D), jnp.float32),
    )(x, partials, partials, W1x, W1a, b1, W2, b2)


def kernel(x, edge_index, edge_weight, W1, b1, W2, b2):
    src = edge_index[1].astype(jnp.int32)
    dst = edge_index[0].astype(jnp.int32)
    half = E // NUM_CORES
    zi = jnp.arange(PAD, dtype=jnp.int32)
    zf = jnp.zeros((PAD,), jnp.float32)
    src_p = jnp.concatenate([src[:half], zi, src[half:], zi])
    dst_p = jnp.concatenate([dst[:half], zi, dst[half:], zi])
    w_p = jnp.concatenate([edge_weight[:half], zf, edge_weight[half:], zf])
    partials = _sc_aggregate(x, src_p, dst_p, w_p)
    W1x = W1[:D]
    W1a = W1[D:]
    return _tc_mlp(x, partials, W1x, W1a, b1.reshape(1, D), W2,
                   b2.reshape(1, D))
